# baseline (speedup 1.0000x reference)
import jax
import jax.numpy as jnp
from jax.experimental import pallas as pl
from jax.experimental.pallas import tpu as pltpu

D_MODEL = 768
N_PAD = 64
BLOCK_T = 4096


def _fused_heads_kernel(
    x_ref, ww_ref, wh_ref, wl_ref, wd_ref,
    bw_ref, bh_ref, bl_ref, bd_ref,
    ow_ref, oh_ref, ol_ref, od_ref,
    w_scratch,
):
    first = jnp.logical_and(pl.program_id(0) == 0, pl.program_id(1) == 0)

    @pl.when(first)
    def _assemble_weights():
        w_scratch[0:16, :] = ww_ref[...]
        w_scratch[16:32, :] = wh_ref[...]
        w_scratch[32:48, :] = wl_ref[...]
        w_scratch[48:63, :] = wd_ref[...]
        w_scratch[63:64, :] = jnp.zeros((1, D_MODEL), jnp.float32)

    res = jax.lax.dot_general(
        w_scratch[...],
        x_ref[0],
        dimension_numbers=(((1,), (1,)), ((), ())),
        preferred_element_type=jnp.float32,
    )
    ow_ref[0] = res[0:16, :] + bw_ref[...]
    oh_ref[0] = res[16:32, :] + bh_ref[...]
    ol_ref[0] = res[32:48, :] + bl_ref[...]
    od_ref[0] = res[48:63, :] + bd_ref[...]


def kernel(conditioning, W_w, b_w, W_h, b_h, W_d, b_d, W_l, b_l):
    B, T, D = conditioning.shape

    grid = (B, T // BLOCK_T)
    ow, oh, ol, od = pl.pallas_call(
        _fused_heads_kernel,
        grid=grid,
        in_specs=[
            pl.BlockSpec((1, BLOCK_T, D), lambda b, j: (b, j, 0)),
            pl.BlockSpec((16, D), lambda b, j: (0, 0)),
            pl.BlockSpec((16, D), lambda b, j: (0, 0)),
            pl.BlockSpec((16, D), lambda b, j: (0, 0)),
            pl.BlockSpec((15, D), lambda b, j: (0, 0)),
            pl.BlockSpec((16, 1), lambda b, j: (0, 0)),
            pl.BlockSpec((16, 1), lambda b, j: (0, 0)),
            pl.BlockSpec((16, 1), lambda b, j: (0, 0)),
            pl.BlockSpec((15, 1), lambda b, j: (0, 0)),
        ],
        out_specs=[
            pl.BlockSpec((1, 16, BLOCK_T), lambda b, j: (b, 0, j)),
            pl.BlockSpec((1, 16, BLOCK_T), lambda b, j: (b, 0, j)),
            pl.BlockSpec((1, 16, BLOCK_T), lambda b, j: (b, 0, j)),
            pl.BlockSpec((1, 15, BLOCK_T), lambda b, j: (b, 0, j)),
        ],
        out_shape=[
            jax.ShapeDtypeStruct((B, 16, T), jnp.float32),
            jax.ShapeDtypeStruct((B, 16, T), jnp.float32),
            jax.ShapeDtypeStruct((B, 16, T), jnp.float32),
            jax.ShapeDtypeStruct((B, 15, T), jnp.float32),
        ],
        scratch_shapes=[pltpu.VMEM((N_PAD, D_MODEL), jnp.float32)],
        compiler_params=pltpu.CompilerParams(
            dimension_semantics=("arbitrary", "arbitrary"),
        ),
    )(
        conditioning, W_w, W_h, W_l, W_d,
        b_w.reshape(16, 1), b_h.reshape(16, 1),
        b_l.reshape(16, 1), b_d.reshape(15, 1),
    )

    return (
        ow.transpose(0, 2, 1),
        oh.transpose(0, 2, 1),
        od.transpose(0, 2, 1),
        ol.transpose(0, 2, 1),
    )


# speedup vs baseline: 1.0671x; 1.0671x over previous
import jax
import jax.numpy as jnp
from jax.experimental import pallas as pl
from jax.experimental.pallas import tpu as pltpu

D_MODEL = 768
N_PAD = 64
BLOCK_T = 4096


def _fused_heads_kernel(
    x_ref, ww_ref, wh_ref, wl_ref, wd_ref, b_ref,
    ow_ref, oh_ref, ol_ref, od_ref,
    w_scratch,
):
    first = jnp.logical_and(pl.program_id(0) == 0, pl.program_id(1) == 0)

    @pl.when(first)
    def _assemble_weights():
        w_scratch[0:16, :] = ww_ref[...]
        w_scratch[16:32, :] = wh_ref[...]
        w_scratch[32:48, :] = wl_ref[...]
        w_scratch[48:63, :] = wd_ref[...]
        w_scratch[63:64, :] = jnp.zeros((1, D_MODEL), jnp.float32)

    res = (
        jax.lax.dot_general(
            w_scratch[...],
            x_ref[0],
            dimension_numbers=(((1,), (1,)), ((), ())),
            preferred_element_type=jnp.float32,
        )
        + b_ref[...]
    )
    ow_ref[0] = res[0:16, :]
    oh_ref[0] = res[16:32, :]
    ol_ref[0] = res[32:48, :]
    od_ref[0] = res[48:63, :]


def kernel(conditioning, W_w, b_w, W_h, b_h, W_d, b_d, W_l, b_l):
    B, T, D = conditioning.shape

    b_cat = jnp.concatenate(
        [b_w, b_h, b_l, b_d, jnp.zeros((1,), jnp.float32)], axis=0
    ).reshape(N_PAD, 1)

    grid = (B, T // BLOCK_T)
    ow, oh, ol, od = pl.pallas_call(
        _fused_heads_kernel,
        grid=grid,
        in_specs=[
            pl.BlockSpec((1, BLOCK_T, D), lambda b, j: (b, j, 0)),
            pl.BlockSpec((16, D), lambda b, j: (0, 0)),
            pl.BlockSpec((16, D), lambda b, j: (0, 0)),
            pl.BlockSpec((16, D), lambda b, j: (0, 0)),
            pl.BlockSpec((15, D), lambda b, j: (0, 0)),
            pl.BlockSpec((N_PAD, 1), lambda b, j: (0, 0)),
        ],
        out_specs=[
            pl.BlockSpec((1, 16, BLOCK_T), lambda b, j: (b, 0, j)),
            pl.BlockSpec((1, 16, BLOCK_T), lambda b, j: (b, 0, j)),
            pl.BlockSpec((1, 16, BLOCK_T), lambda b, j: (b, 0, j)),
            pl.BlockSpec((1, 15, BLOCK_T), lambda b, j: (b, 0, j)),
        ],
        out_shape=[
            jax.ShapeDtypeStruct((B, 16, T), jnp.float32),
            jax.ShapeDtypeStruct((B, 16, T), jnp.float32),
            jax.ShapeDtypeStruct((B, 16, T), jnp.float32),
            jax.ShapeDtypeStruct((B, 15, T), jnp.float32),
        ],
        scratch_shapes=[pltpu.VMEM((N_PAD, D_MODEL), jnp.float32)],
        compiler_params=pltpu.CompilerParams(
            dimension_semantics=("arbitrary", "arbitrary"),
        ),
    )(conditioning, W_w, W_h, W_l, W_d, b_cat)

    return (
        ow.transpose(0, 2, 1),
        oh.transpose(0, 2, 1),
        od.transpose(0, 2, 1),
        ol.transpose(0, 2, 1),
    )
